# Initial kernel scaffold; baseline (speedup 1.0000x reference)
import jax
import jax.numpy as jnp
from jax.experimental import pallas as pl


def kernel(sim_values, sim_rows, sim_cols):
    raise NotImplementedError("write your pallas kernel here")


# speedup vs baseline: 34.2420x; 34.2420x over previous
import functools

import jax
import jax.numpy as jnp
from jax import lax
from jax.experimental import pallas as pl
from jax.experimental.pallas import tpu as pltpu
from jax.experimental.pallas import tpu_sc as plsc

N1 = 4096
N2 = 4096
L = 16
NC = 2
NS = 16
NW = NC * NS
BIGC = 1 << 30
NEGV = -1.0


_TAKE_DNUMS = lax.GatherDimensionNumbers(
    offset_dims=(), collapsed_slice_dims=(0,), start_index_map=(0,))


def _take(x, idx):
  return lax.gather(x, idx[:, None], _TAKE_DNUMS, slice_sizes=(1,),
                    mode=lax.GatherScatterMode.PROMISE_IN_BOUNDS)


def _lex_improves(v_new, c_new, v_old, c_old):
  return (v_new > v_old) | ((v_new == v_old) & (c_new < c_old))


def _partial_body(nvecs, vals_hbm, rows_hbm, cols_hbm, pv_hbm, pc_hbm,
                  vals_v, rows_v, cols_v, accv, accc):
  wid = lax.axis_index("s") * NC + lax.axis_index("c")
  ch = nvecs * L
  base = wid * ch
  pltpu.sync_copy(vals_hbm.at[pl.ds(base, ch)], vals_v)
  pltpu.sync_copy(rows_hbm.at[pl.ds(base, ch)], rows_v)
  pltpu.sync_copy(cols_hbm.at[pl.ds(base, ch)], cols_v)

  iota = lax.iota(jnp.int32, L)

  def init(j, _):
    accv[pl.ds(j * L, L)] = jnp.full((L,), NEGV, jnp.float32)
    accc[pl.ds(j * L, L)] = jnp.full((L,), BIGC, jnp.int32)
    return 0

  lax.fori_loop(0, N1 // L, init, 0)

  def step(i, _):
    r = rows_v[pl.ds(i * L, L)]
    v = vals_v[pl.ds(i * L, L)]
    c = cols_v[pl.ds(i * L, L)]
    for d in (1, 2, 4, 8):
      idx = jnp.maximum(iota - d, 0)
      rd = _take(r, idx)
      vd = _take(v, idx)
      cd = _take(c, idx)
      take_new = (rd == r) & _lex_improves(vd, cd, v, c)
      v = jnp.where(take_new, vd, v)
      c = jnp.where(take_new, cd, c)
    rn = _take(r, jnp.minimum(iota + 1, L - 1))
    last = (rn != r) | (iota == L - 1)
    av = plsc.load_gather(accv, [r], mask=last)
    ac = plsc.load_gather(accc, [r], mask=last)
    upd = last & _lex_improves(v, c, av, ac)
    plsc.store_scatter(accv, [r], v, mask=upd)
    plsc.store_scatter(accc, [r], c, mask=upd)
    return 0

  lax.fori_loop(0, nvecs, step, 0)

  pltpu.sync_copy(accv, pv_hbm.at[wid])
  pltpu.sync_copy(accc, pc_hbm.at[wid])


def _combine_body(pv_hbm, pc_hbm, out_v_hbm, out_c_hbm,
                  bufv, bufc, resv, resc):
  wid = lax.axis_index("s") * NC + lax.axis_index("c")
  rows_per_w = N1 // NW
  base = wid * rows_per_w
  pltpu.sync_copy(pv_hbm.at[:, pl.ds(base, rows_per_w)], bufv)
  pltpu.sync_copy(pc_hbm.at[:, pl.ds(base, rows_per_w)], bufc)

  for t in range(rows_per_w // L):
    sl = pl.ds(t * L, L)

    def red(w, carry):
      mv, mc = carry
      xv = bufv[w, sl]
      xc = bufc[w, sl]
      upd = _lex_improves(xv, xc, mv, mc)
      return jnp.where(upd, xv, mv), jnp.where(upd, xc, mc)

    mv, mc = lax.fori_loop(1, NW, red, (bufv[0, sl], bufc[0, sl]))
    resv[sl] = jnp.maximum(mv, 0.0)
    resc[sl] = jnp.where(mv > 0.0, mc, 0)

  pltpu.sync_copy(resv, out_v_hbm.at[pl.ds(base, rows_per_w)])
  pltpu.sync_copy(resc, out_c_hbm.at[pl.ds(base, rows_per_w)])


@jax.jit
def kernel(sim_values, sim_rows, sim_cols):
  nnz = sim_values.shape[0]
  nvecs = -(-nnz // (NW * L))
  pad = NW * L * nvecs - nnz
  vals = jnp.concatenate(
      [sim_values, jnp.full((pad,), NEGV, sim_values.dtype)])
  rows = jnp.concatenate(
      [sim_rows.astype(jnp.int32), jnp.full((pad,), N1 - 1, jnp.int32)])
  cols = jnp.concatenate(
      [sim_cols.astype(jnp.int32), jnp.full((pad,), BIGC, jnp.int32)])

  mesh = plsc.VectorSubcoreMesh(
      core_axis_name="c", subcore_axis_name="s", num_cores=NC,
      num_subcores=NS)

  ch = nvecs * L
  params = pltpu.CompilerParams(needs_layout_passes=False)
  partial = pl.kernel(
      functools.partial(_partial_body, nvecs),
      compiler_params=params,
      out_type=(
          jax.ShapeDtypeStruct((NW, N1), jnp.float32),
          jax.ShapeDtypeStruct((NW, N1), jnp.int32),
      ),
      mesh=mesh,
      scratch_types=[
          pltpu.VMEM((ch,), jnp.float32),
          pltpu.VMEM((ch,), jnp.int32),
          pltpu.VMEM((ch,), jnp.int32),
          pltpu.VMEM((N1,), jnp.float32),
          pltpu.VMEM((N1,), jnp.int32),
      ],
  )
  pv, pc = partial(vals, rows, cols)

  combine = pl.kernel(
      _combine_body,
      compiler_params=params,
      out_type=(
          jax.ShapeDtypeStruct((N1,), jnp.float32),
          jax.ShapeDtypeStruct((N1,), jnp.int32),
      ),
      mesh=mesh,
      scratch_types=[
          pltpu.VMEM((NW, N1 // NW), jnp.float32),
          pltpu.VMEM((NW, N1 // NW), jnp.int32),
          pltpu.VMEM((N1 // NW,), jnp.float32),
          pltpu.VMEM((N1 // NW,), jnp.int32),
      ],
  )
  scores, indices = combine(pv, pc)
  return scores, indices


# speedup vs baseline: 37.9279x; 1.1076x over previous
import functools

import jax
import jax.numpy as jnp
from jax import lax
from jax.experimental import pallas as pl
from jax.experimental.pallas import tpu as pltpu
from jax.experimental.pallas import tpu_sc as plsc

N1 = 4096
N2 = 4096
RC_BITS = 12
L = 16
NC = 2
NS = 16
NW = NC * NS
BIGC = 1 << 30
SENT_RC = (1 << 24) - 1
NEGV = -1.0

_TAKE_DNUMS = lax.GatherDimensionNumbers(
    offset_dims=(), collapsed_slice_dims=(0,), start_index_map=(0,))


def _take(x, idx):
  return lax.gather(x, idx[:, None], _TAKE_DNUMS, slice_sizes=(1,),
                    mode=lax.GatherScatterMode.PROMISE_IN_BOUNDS)


def _lex_improves(v_new, rc_new, v_old, rc_old):
  return (v_new > v_old) | ((v_new == v_old) & (rc_new < rc_old))


def _sc_partial_body(nvecs, nnz, vals_hbm, rc_hbm, pv_hbm, pc_hbm,
                     vals_v, rc_v, accv, accrc):
  wid = lax.axis_index("s") * NC + lax.axis_index("c")
  ch = nvecs * L
  base = wid * ch
  short_len = nnz - (NW - 1) * ch

  def full_dma():
    pltpu.sync_copy(vals_hbm.at[pl.ds(base, ch)], vals_v.at[pl.ds(0, ch)])
    pltpu.sync_copy(rc_hbm.at[pl.ds(base, ch)], rc_v.at[pl.ds(0, ch)])

  def short_dma():
    pltpu.sync_copy(vals_hbm.at[pl.ds(base, short_len)],
                    vals_v.at[pl.ds(0, short_len)])
    pltpu.sync_copy(rc_hbm.at[pl.ds(base, short_len)],
                    rc_v.at[pl.ds(0, short_len)])
    for off in range(short_len, ch, L):
      vals_v[pl.ds(off, L)] = jnp.full((L,), NEGV, jnp.float32)
      rc_v[pl.ds(off, L)] = jnp.full((L,), SENT_RC, jnp.int32)

  lax.cond(wid == NW - 1, short_dma, full_dma)

  iota = lax.iota(jnp.int32, L)

  def init(j, _):
    accv[pl.ds(j * L, L)] = jnp.full((L,), NEGV, jnp.float32)
    accrc[pl.ds(j * L, L)] = jnp.full((L,), BIGC, jnp.int32)
    return 0

  lax.fori_loop(0, N1 // L, init, 0)

  idxs = [jnp.maximum(iota - d, 0) for d in (1, 2, 4, 8)]
  idx_next = jnp.minimum(iota + 1, L - 1)

  def step(i, _):
    v = vals_v[pl.ds(i * L, L)]
    rc = rc_v[pl.ds(i * L, L)]
    r = lax.shift_right_logical(rc, RC_BITS)
    for idx in idxs:
      vd = _take(v, idx)
      rcd = _take(rc, idx)
      same = lax.shift_right_logical(rcd, RC_BITS) == r
      tk = same & _lex_improves(vd, rcd, v, rc)
      v = jnp.where(tk, vd, v)
      rc = jnp.where(tk, rcd, rc)
    rn = lax.shift_right_logical(_take(rc, idx_next), RC_BITS)
    last = (rn != r) | (iota == L - 1)
    av = plsc.load_gather(accv, [r], mask=last)
    arc = plsc.load_gather(accrc, [r], mask=last)
    upd = last & _lex_improves(v, rc, av, arc)
    plsc.store_scatter(accv, [r], v, mask=upd)
    plsc.store_scatter(accrc, [r], rc, mask=upd)
    return 0

  lax.fori_loop(0, nvecs, step, 0)

  pltpu.sync_copy(accv, pv_hbm.at[wid])
  pltpu.sync_copy(accrc, pc_hbm.at[wid])


def _tc_combine_body(pv_ref, prc_ref, outv_ref, outc_ref):
  bv = pv_ref[0, :]
  brc = prc_ref[0, :]
  for w in range(1, NW):
    xv = pv_ref[w, :]
    xrc = prc_ref[w, :]
    upd = _lex_improves(xv, xrc, bv, brc)
    bv = jnp.where(upd, xv, bv)
    brc = jnp.where(upd, xrc, brc)
  outv_ref[:] = jnp.maximum(bv, 0.0)
  outc_ref[:] = jnp.where(bv > 0.0, brc & (N2 - 1), 0)


@jax.jit
def kernel(sim_values, sim_rows, sim_cols):
  nnz = sim_values.shape[0]
  nvecs = -(-nnz // (NW * L))
  rc = lax.shift_left(sim_rows.astype(jnp.int32), RC_BITS) | (
      sim_cols.astype(jnp.int32))

  mesh = plsc.VectorSubcoreMesh(
      core_axis_name="c", subcore_axis_name="s", num_cores=NC,
      num_subcores=NS)

  ch = nvecs * L
  partial = pl.kernel(
      functools.partial(_sc_partial_body, nvecs, nnz),
      compiler_params=pltpu.CompilerParams(needs_layout_passes=False),
      out_type=(
          jax.ShapeDtypeStruct((NW, N1), jnp.float32),
          jax.ShapeDtypeStruct((NW, N1), jnp.int32),
      ),
      mesh=mesh,
      scratch_types=[
          pltpu.VMEM((ch + L,), jnp.float32),
          pltpu.VMEM((ch + L,), jnp.int32),
          pltpu.VMEM((N1,), jnp.float32),
          pltpu.VMEM((N1,), jnp.int32),
      ],
  )
  pv, prc = partial(sim_values, rc)

  scores, indices = pl.pallas_call(
      _tc_combine_body,
      out_shape=(
          jax.ShapeDtypeStruct((N1,), jnp.float32),
          jax.ShapeDtypeStruct((N1,), jnp.int32),
      ),
  )(pv, prc)
  return scores, indices


# speedup vs baseline: 39.0897x; 1.0306x over previous
import functools

import jax
import jax.numpy as jnp
from jax import lax
from jax.experimental import pallas as pl
from jax.experimental.pallas import tpu as pltpu
from jax.experimental.pallas import tpu_sc as plsc

N1 = 4096
N2 = 4096
RC_BITS = 12
L = 16
NC = 2
NS = 16
NW = NC * NS
BIGC = 1 << 30
SENT_RC = (1 << 24) - 1
NEGV = -1.0

_TAKE_DNUMS = lax.GatherDimensionNumbers(
    offset_dims=(), collapsed_slice_dims=(0,), start_index_map=(0,))


def _take(x, idx):
  return lax.gather(x, idx[:, None], _TAKE_DNUMS, slice_sizes=(1,),
                    mode=lax.GatherScatterMode.PROMISE_IN_BOUNDS)


def _lex_improves(v_new, rc_new, v_old, rc_old):
  return (v_new > v_old) | ((v_new == v_old) & (rc_new < rc_old))


def _sc_partial_body(nvecs, vals_hbm, rc_hbm, pv_hbm, pc_hbm,
                     vals_v, rc_v, accv, accrc, sem1, sem2):
  wid = lax.axis_index("s") * NC + lax.axis_index("c")
  ch = nvecs * L
  base = wid * ch
  h1 = pltpu.async_copy(vals_hbm.at[pl.ds(base, ch)], vals_v, sem1)
  h2 = pltpu.async_copy(rc_hbm.at[pl.ds(base, ch)], rc_v, sem2)

  iota = lax.iota(jnp.int32, L)

  def init(j, _):
    accv[pl.ds(j * L, L)] = jnp.full((L,), NEGV, jnp.float32)
    accrc[pl.ds(j * L, L)] = jnp.full((L,), BIGC, jnp.int32)
    return 0

  lax.fori_loop(0, N1 // L, init, 0)
  h1.wait()
  h2.wait()

  idxs = [jnp.maximum(iota - d, 0) for d in (1, 2, 4, 8)]
  idx_next = jnp.minimum(iota + 1, L - 1)

  def one_vreg(i):
    v = vals_v[pl.ds(i * L, L)]
    rc = rc_v[pl.ds(i * L, L)]
    r = lax.shift_right_logical(rc, RC_BITS)
    for idx in idxs:
      vd = _take(v, idx)
      rcd = _take(rc, idx)
      same = lax.shift_right_logical(rcd, RC_BITS) == r
      tk = same & _lex_improves(vd, rcd, v, rc)
      v = jnp.where(tk, vd, v)
      rc = jnp.where(tk, rcd, rc)
    rn = lax.shift_right_logical(_take(rc, idx_next), RC_BITS)
    last = (rn != r) | (iota == L - 1)
    av = plsc.load_gather(accv, [r], mask=last)
    arc = plsc.load_gather(accrc, [r], mask=last)
    upd = last & _lex_improves(v, rc, av, arc)
    plsc.store_scatter(accv, [r], v, mask=upd)
    plsc.store_scatter(accrc, [r], rc, mask=upd)

  def step(i, _):
    one_vreg(2 * i)
    one_vreg(2 * i + 1)
    return 0

  lax.fori_loop(0, nvecs // 2, step, 0)

  pltpu.sync_copy(accv, pv_hbm.at[wid])
  pltpu.sync_copy(accrc, pc_hbm.at[wid])


def _tc_combine_body(pv_ref, prc_ref, outv_ref, outc_ref):
  bv = pv_ref[0, :]
  brc = prc_ref[0, :]
  for w in range(1, NW):
    xv = pv_ref[w, :]
    xrc = prc_ref[w, :]
    upd = _lex_improves(xv, xrc, bv, brc)
    bv = jnp.where(upd, xv, bv)
    brc = jnp.where(upd, xrc, brc)
  outv_ref[:] = jnp.maximum(bv, 0.0)
  outc_ref[:] = jnp.where(bv > 0.0, brc & (N2 - 1), 0)


@jax.jit
def kernel(sim_values, sim_rows, sim_cols):
  nnz = sim_values.shape[0]
  nvecs = -(-nnz // (NW * L))
  pad = NW * L * nvecs - nnz
  vals = jnp.concatenate(
      [sim_values, jnp.full((pad,), NEGV, sim_values.dtype)])
  rc_raw = lax.shift_left(sim_rows.astype(jnp.int32), RC_BITS) | (
      sim_cols.astype(jnp.int32))
  rc = jnp.concatenate([rc_raw, jnp.full((pad,), SENT_RC, jnp.int32)])

  mesh = plsc.VectorSubcoreMesh(
      core_axis_name="c", subcore_axis_name="s", num_cores=NC,
      num_subcores=NS)

  ch = nvecs * L
  partial = pl.kernel(
      functools.partial(_sc_partial_body, nvecs),
      compiler_params=pltpu.CompilerParams(needs_layout_passes=False),
      out_type=(
          jax.ShapeDtypeStruct((NW, N1), jnp.float32),
          jax.ShapeDtypeStruct((NW, N1), jnp.int32),
      ),
      mesh=mesh,
      scratch_types=[
          pltpu.VMEM((ch,), jnp.float32),
          pltpu.VMEM((ch,), jnp.int32),
          pltpu.VMEM((N1,), jnp.float32),
          pltpu.VMEM((N1,), jnp.int32),
          pltpu.SemaphoreType.DMA,
          pltpu.SemaphoreType.DMA,
      ],
  )
  pv, prc = partial(vals, rc)

  scores, indices = pl.pallas_call(
      _tc_combine_body,
      out_shape=(
          jax.ShapeDtypeStruct((N1,), jnp.float32),
          jax.ShapeDtypeStruct((N1,), jnp.int32),
      ),
  )(pv, prc)
  return scores, indices


# speedup vs baseline: 42.4624x; 1.0863x over previous
import functools

import jax
import jax.numpy as jnp
from jax import lax
from jax.experimental import pallas as pl
from jax.experimental.pallas import tpu as pltpu
from jax.experimental.pallas import tpu_sc as plsc

N1 = 4096
N2 = 4096
RC_BITS = 12
L = 16
NC = 2
NS = 16
NW = NC * NS
D0 = 8
BIGC = 1 << 30
SENT_RC = (1 << 24) - 1
BREAK_RC = -1
NEGV = -1.0

_TAKE_DNUMS = lax.GatherDimensionNumbers(
    offset_dims=(), collapsed_slice_dims=(0,), start_index_map=(0,))


def _take(x, idx):
  return lax.gather(x, idx[:, None], _TAKE_DNUMS, slice_sizes=(1,),
                    mode=lax.GatherScatterMode.PROMISE_IN_BOUNDS)


def _lex_improves(v_new, rc_new, v_old, rc_old):
  return (v_new > v_old) | ((v_new == v_old) & (rc_new < rc_old))


def _row(rc):
  return lax.shift_right_logical(rc, RC_BITS)


def _sc_partial_body(nvecs, nnz, vals_hbm, rc_hbm, pv_hbm, pc_hbm,
                     vals_v, rc_v, accv, accrc, sem1, sem2):
  wid = lax.axis_index("s") * NC + lax.axis_index("c")
  ch = nvecs * L
  base_a = (nnz - ch) & ~7
  tail_at = base_a + ch
  rem = nnz - tail_at
  is_last = wid == NW - 1
  base = jnp.minimum(wid * ch, base_a)

  vals_v[pl.ds(0, L)] = jnp.full((L,), NEGV, jnp.float32)
  rc_v[pl.ds(0, L)] = jnp.full((L,), SENT_RC, jnp.int32)
  vals_v[pl.ds(D0 + ch, L)] = jnp.full((L,), NEGV, jnp.float32)
  rc_v[pl.ds(D0 + ch, L)] = jnp.full((L,), BREAK_RC, jnp.int32)

  h1 = pltpu.async_copy(vals_hbm.at[pl.ds(base, ch)],
                        vals_v.at[pl.ds(D0, ch)], sem1)
  h2 = pltpu.async_copy(rc_hbm.at[pl.ds(base, ch)],
                        rc_v.at[pl.ds(D0, ch)], sem2)

  if rem:
    @pl.when(is_last)
    def _tail():
      rc_v[pl.ds(D0 + ch, L)] = jnp.full((L,), SENT_RC, jnp.int32)
      vals_v[pl.ds(D0 + ch + L, L)] = jnp.full((L,), NEGV, jnp.float32)
      rc_v[pl.ds(D0 + ch + L, L)] = jnp.full((L,), BREAK_RC, jnp.int32)
      pltpu.sync_copy(vals_hbm.at[pl.ds(tail_at, rem)],
                      vals_v.at[pl.ds(D0 + ch, rem)])
      pltpu.sync_copy(rc_hbm.at[pl.ds(tail_at, rem)],
                      rc_v.at[pl.ds(D0 + ch, rem)])

  def init(j, _):
    accv[pl.ds(j * L, L)] = jnp.full((L,), NEGV, jnp.float32)
    accrc[pl.ds(j * L, L)] = jnp.full((L,), BIGC, jnp.int32)
    return 0

  lax.fori_loop(0, N1 // L, init, 0)
  h1.wait()
  h2.wait()

  iota = lax.iota(jnp.int32, L)
  idxs = [jnp.maximum(iota - d, 0) for d in (2, 4, 8)]

  def one_vreg(i):
    o = D0 + i * L
    v = vals_v[pl.ds(o, L)]
    rc = rc_v[pl.ds(o, L)]
    r = _row(rc)
    vd = vals_v[pl.ds(o - 1, L)]
    rcd = rc_v[pl.ds(o - 1, L)]
    tk = (_row(rcd) == r) & _lex_improves(vd, rcd, v, rc)
    v = jnp.where(tk, vd, v)
    rc = jnp.where(tk, rcd, rc)
    for idx in idxs:
      vd = _take(v, idx)
      rcd = _take(rc, idx)
      tk = (_row(rcd) == r) & _lex_improves(vd, rcd, v, rc)
      v = jnp.where(tk, vd, v)
      rc = jnp.where(tk, rcd, rc)
    last = (_row(rc_v[pl.ds(o + 1, L)]) != r) | (iota == L - 1)
    av = plsc.load_gather(accv, [r], mask=last)
    arc = plsc.load_gather(accrc, [r], mask=last)
    upd = last & _lex_improves(v, rc, av, arc)
    plsc.store_scatter(accv, [r], v, mask=upd)
    plsc.store_scatter(accrc, [r], rc, mask=upd)

  def step(i, _):
    one_vreg(2 * i)
    one_vreg(2 * i + 1)
    return 0

  lax.fori_loop(0, nvecs // 2, step, 0)
  if rem:
    @pl.when(is_last)
    def _tail_vreg():
      one_vreg(nvecs)

  pltpu.sync_copy(accv, pv_hbm.at[wid])
  pltpu.sync_copy(accrc, pc_hbm.at[wid])


def _tc_combine_body(pv_ref, prc_ref, outv_ref, outc_ref):
  bv = pv_ref[0, :]
  brc = prc_ref[0, :]
  for w in range(1, NW):
    xv = pv_ref[w, :]
    xrc = prc_ref[w, :]
    upd = _lex_improves(xv, xrc, bv, brc)
    bv = jnp.where(upd, xv, bv)
    brc = jnp.where(upd, xrc, brc)
  outv_ref[:] = jnp.maximum(bv, 0.0)
  outc_ref[:] = jnp.where(bv > 0.0, brc & (N2 - 1), 0)


@jax.jit
def kernel(sim_values, sim_rows, sim_cols):
  nnz = sim_values.shape[0]
  nvecs = -(-nnz // (NW * L))
  rc = lax.shift_left(sim_rows.astype(jnp.int32), RC_BITS) | (
      sim_cols.astype(jnp.int32))

  mesh = plsc.VectorSubcoreMesh(
      core_axis_name="c", subcore_axis_name="s", num_cores=NC,
      num_subcores=NS)

  ch = nvecs * L
  partial = pl.kernel(
      functools.partial(_sc_partial_body, nvecs, nnz),
      compiler_params=pltpu.CompilerParams(needs_layout_passes=False),
      out_type=(
          jax.ShapeDtypeStruct((NW, N1), jnp.float32),
          jax.ShapeDtypeStruct((NW, N1), jnp.int32),
      ),
      mesh=mesh,
      scratch_types=[
          pltpu.VMEM((D0 + ch + 2 * L,), jnp.float32),
          pltpu.VMEM((D0 + ch + 2 * L,), jnp.int32),
          pltpu.VMEM((N1,), jnp.float32),
          pltpu.VMEM((N1,), jnp.int32),
          pltpu.SemaphoreType.DMA,
          pltpu.SemaphoreType.DMA,
      ],
  )
  pv, prc = partial(sim_values, rc)

  scores, indices = pl.pallas_call(
      _tc_combine_body,
      out_shape=(
          jax.ShapeDtypeStruct((N1,), jnp.float32),
          jax.ShapeDtypeStruct((N1,), jnp.int32),
      ),
  )(pv, prc)
  return scores, indices
